# baseline (speedup 1.0000x reference)
import jax
import jax.numpy as jnp
from jax.experimental import pallas as pl
from jax.experimental.pallas import tpu as pltpu
from jax.experimental.pallas import tpu_sc as plsc


def kernel(input_or_shape, pos_table):
    batch, seq_len = input_or_shape.shape
    max_pos, hidden = pos_table.shape
    dtype = pos_table.dtype

    width = 256
    sub = hidden // width
    tab = pos_table.reshape(max_pos * sub, width)

    position_ids = jnp.clip(jnp.arange(seq_len, dtype=jnp.int32), 0, max_pos - 1)
    sub_idx = position_ids[:, None] * sub + jnp.arange(sub, dtype=jnp.int32)[None, :]
    flat_idx = jnp.broadcast_to(
        sub_idx.reshape(1, seq_len * sub), (batch, seq_len * sub)
    ).reshape(1, batch * seq_len * sub)
    n_idx = batch * seq_len * sub

    window = 128
    mesh = plsc.VectorSubcoreMesh(core_axis_name="c", subcore_axis_name="s")

    @pl.kernel(
        out_type=jax.ShapeDtypeStruct((n_idx, width), dtype),
        mesh=mesh,
    )
    def sc_gather(tab_hbm, i_hbm, o_hbm):
        def body(i_vmem, o_vmem):
            pltpu.sync_copy(tab_hbm.at[i_vmem.at[0]], o_vmem)

        pltpu.emit_pipeline(
            body,
            grid=(n_idx // window,),
            in_specs=[pl.BlockSpec((1, window), lambda i: (0, i))],
            out_specs=[pl.BlockSpec((window, width), lambda i: (i, 0))],
            core_axis_name=("c", "s"),
            dimension_semantics=(pltpu.PARALLEL,),
        )(i_hbm, o_hbm)

    out = sc_gather(tab, flat_idx)
    return out.reshape(batch, seq_len, hidden)


# speedup vs baseline: 5.7954x; 5.7954x over previous
import jax
import jax.numpy as jnp
from jax.experimental import pallas as pl
from jax.experimental.pallas import tpu as pltpu


def kernel(input_or_shape, pos_table):
    batch, seq_len = input_or_shape.shape
    max_pos, hidden = pos_table.shape
    dtype = pos_table.dtype

    block_s = 512
    n = seq_len // block_s
    nbuf = 4

    def body(tab_hbm, out_hbm, buf, insem, outsem):
        def in_copy(i):
            return pltpu.make_async_copy(
                tab_hbm.at[pl.ds(i * block_s, block_s), :],
                buf.at[i % nbuf],
                insem.at[i % nbuf],
            )

        def out_copies(i):
            return [
                pltpu.make_async_copy(
                    buf.at[i % nbuf],
                    out_hbm.at[b, pl.ds(i * block_s, block_s), :],
                    outsem.at[i % nbuf, b],
                )
                for b in range(batch)
            ]

        for i in range(nbuf):
            in_copy(i).start()
        for i in range(n):
            nxt = i + 1
            if nxt >= nbuf and nxt < n:
                for cp in out_copies(nxt - nbuf):
                    cp.wait()
                in_copy(nxt).start()
            in_copy(i).wait()
            for cp in out_copies(i):
                cp.start()
        for i in range(n - nbuf, n):
            for cp in out_copies(i):
                cp.wait()

    return pl.pallas_call(
        body,
        in_specs=[pl.BlockSpec(memory_space=pl.ANY)],
        out_specs=pl.BlockSpec(memory_space=pl.ANY),
        out_shape=jax.ShapeDtypeStruct((batch, seq_len, hidden), dtype),
        scratch_shapes=[
            pltpu.VMEM((nbuf, block_s, hidden), dtype),
            pltpu.SemaphoreType.DMA((nbuf,)),
            pltpu.SemaphoreType.DMA((nbuf, batch)),
        ],
    )(pos_table)


# speedup vs baseline: 5.9768x; 1.0313x over previous
import jax
import jax.numpy as jnp
from jax.experimental import pallas as pl
from jax.experimental.pallas import tpu as pltpu


def kernel(input_or_shape, pos_table):
    batch, seq_len = input_or_shape.shape
    max_pos, hidden = pos_table.shape
    dtype = pos_table.dtype

    block_s = 1024
    n = seq_len // block_s
    nbuf = 4

    def body(tab_hbm, out_hbm, buf, insem, outsem):
        def in_copy(i):
            return pltpu.make_async_copy(
                tab_hbm.at[pl.ds(i * block_s, block_s), :],
                buf.at[i % nbuf],
                insem.at[i % nbuf],
            )

        def out_copies(i):
            return [
                pltpu.make_async_copy(
                    buf.at[i % nbuf],
                    out_hbm.at[b, pl.ds(i * block_s, block_s), :],
                    outsem.at[i % nbuf, b],
                )
                for b in range(batch)
            ]

        for i in range(nbuf):
            in_copy(i).start()
        for i in range(n):
            nxt = i + 1
            if nxt >= nbuf and nxt < n:
                for cp in out_copies(nxt - nbuf):
                    cp.wait()
                in_copy(nxt).start()
            in_copy(i).wait()
            for cp in out_copies(i):
                cp.start()
        for i in range(n - nbuf, n):
            for cp in out_copies(i):
                cp.wait()

    return pl.pallas_call(
        body,
        in_specs=[pl.BlockSpec(memory_space=pl.ANY)],
        out_specs=pl.BlockSpec(memory_space=pl.ANY),
        out_shape=jax.ShapeDtypeStruct((batch, seq_len, hidden), dtype),
        scratch_shapes=[
            pltpu.VMEM((nbuf, block_s, hidden), dtype),
            pltpu.SemaphoreType.DMA((nbuf,)),
            pltpu.SemaphoreType.DMA((nbuf, batch)),
        ],
    )(pos_table)


# speedup vs baseline: 6.0601x; 1.0139x over previous
import jax
import jax.numpy as jnp
from jax.experimental import pallas as pl
from jax.experimental.pallas import tpu as pltpu


def kernel(input_or_shape, pos_table):
    batch, seq_len = input_or_shape.shape
    max_pos, hidden = pos_table.shape
    dtype = pos_table.dtype

    block_s = 2048
    n = seq_len // block_s
    nbuf = 4

    def body(tab_hbm, out_hbm, buf, insem, outsem):
        def in_copy(i):
            return pltpu.make_async_copy(
                tab_hbm.at[pl.ds(i * block_s, block_s), :],
                buf.at[i % nbuf],
                insem.at[i % nbuf],
            )

        def out_copies(i):
            return [
                pltpu.make_async_copy(
                    buf.at[i % nbuf],
                    out_hbm.at[b, pl.ds(i * block_s, block_s), :],
                    outsem.at[i % nbuf, b],
                )
                for b in range(batch)
            ]

        for i in range(nbuf):
            in_copy(i).start()
        for i in range(n):
            nxt = i + 1
            if nxt >= nbuf and nxt < n:
                for cp in out_copies(nxt - nbuf):
                    cp.wait()
                in_copy(nxt).start()
            in_copy(i).wait()
            for cp in out_copies(i):
                cp.start()
        for i in range(n - nbuf, n):
            for cp in out_copies(i):
                cp.wait()

    return pl.pallas_call(
        body,
        in_specs=[pl.BlockSpec(memory_space=pl.ANY)],
        out_specs=pl.BlockSpec(memory_space=pl.ANY),
        out_shape=jax.ShapeDtypeStruct((batch, seq_len, hidden), dtype),
        scratch_shapes=[
            pltpu.VMEM((nbuf, block_s, hidden), dtype),
            pltpu.SemaphoreType.DMA((nbuf,)),
            pltpu.SemaphoreType.DMA((nbuf, batch)),
        ],
    )(pos_table)
